# baseline (speedup 1.0000x reference)
import functools

import jax
import jax.numpy as jnp
from jax import lax
from jax.experimental import pallas as pl
from jax.experimental.pallas import tpu as pltpu
from jax.experimental.pallas import tpu_sc as plsc

C, H, W = 128, 512, 512
N = 200000
NW = 32
B = 96
NT = 66
NCHUNK = NW * NT
N_PAD = NCHUNK * B
PW = NT * B
G = B // 16


def _body(ur_hbm, vr_hbm, fm_hbm, out_hbm,
          u_all, v_all, i4a, i4b, r4a, r4b, gsem, osem):
    cid = lax.axis_index("c")
    sid = lax.axis_index("s")
    wid = sid * 2 + cid
    pltpu.sync_copy(ur_hbm.at[wid], u_all)
    pltpu.sync_copy(vr_hbm.at[wid], v_all)

    i4 = (i4a, i4b)
    r4 = (r4a, r4b)

    def fire_gather(b, k):
        koff = pl.multiple_of(k * B, B)
        for g in range(G):
            sl = pl.ds(koff + g * 16, 16)
            uu = u_all[sl]
            vv = v_all[sl]
            b00 = vv.astype(jnp.int32) * W + uu.astype(jnp.int32)
            i4[b][pl.ds(g * 16, 16)] = b00
            i4[b][pl.ds(B + g * 16, 16)] = b00 + 1
            i4[b][pl.ds(2 * B + g * 16, 16)] = b00 + W
            i4[b][pl.ds(3 * B + g * 16, 16)] = b00 + (W + 1)
        pltpu.async_copy(fm_hbm.at[i4[b]], r4[b], gsem.at[b])

    def combine_store(b, k):
        pltpu.make_async_copy(fm_hbm.at[i4[b]], r4[b], gsem.at[b]).wait()
        koff = pl.multiple_of(k * B, B)

        def gb(g, _):
            goff = pl.multiple_of(g * 16, 16)
            uu = u_all[pl.ds(koff + goff, 16)]
            vv = v_all[pl.ds(koff + goff, 16)]
            du = uu - uu.astype(jnp.int32).astype(jnp.float32)
            dv = vv - vv.astype(jnp.int32).astype(jnp.float32)
            omu = 1.0 - du
            omv = 1.0 - dv
            w00 = omv * omu
            w01 = omv * du
            w10 = dv * omu
            w11 = dv * du
            for j in range(0, 16, 2):
                s00 = jnp.full((16,), w00[j], jnp.float32)
                s01 = jnp.full((16,), w01[j], jnp.float32)
                s10 = jnp.full((16,), w10[j], jnp.float32)
                s11 = jnp.full((16,), w11[j], jnp.float32)
                t00 = jnp.full((16,), w00[j + 1], jnp.float32)
                t01 = jnp.full((16,), w01[j + 1], jnp.float32)
                t10 = jnp.full((16,), w10[j + 1], jnp.float32)
                t11 = jnp.full((16,), w11[j + 1], jnp.float32)
                p = goff + j
                q = p + 1
                for cg in range(C // 16):
                    cs = pl.ds(cg * 16, 16)
                    res = (s00 * r4[b][p, cs] + s01 * r4[b][p + B, cs]
                           + s10 * r4[b][p + 2 * B, cs] + s11 * r4[b][p + 3 * B, cs])
                    rqs = (t00 * r4[b][q, cs] + t01 * r4[b][q + B, cs]
                           + t10 * r4[b][q + 2 * B, cs] + t11 * r4[b][q + 3 * B, cs])
                    r4[b][p, cs] = res
                    r4[b][q, cs] = rqs
            return _

        lax.fori_loop(0, G, gb, 0)
        base = (wid + k * NW) * B
        pltpu.async_copy(r4[b].at[pl.ds(0, B)], out_hbm.at[pl.ds(base, B)],
                         osem.at[b])

    def drain_out(b, k):
        base = (wid + k * NW) * B
        pltpu.make_async_copy(r4[b].at[pl.ds(0, B)],
                              out_hbm.at[pl.ds(base, B)], osem.at[b]).wait()

    fire_gather(0, 0)

    def step2(m, carry):
        k0 = m * 2
        fire_gather(1, k0 + 1)

        @pl.when(m >= 1)
        def _():
            drain_out(0, k0 - 2)
        combine_store(0, k0)

        @pl.when(m + 1 <= NT // 2 - 1)
        def _():
            fire_gather(0, k0 + 2)

        @pl.when(m >= 1)
        def _():
            drain_out(1, k0 - 1)
        combine_store(1, k0 + 1)
        return carry

    lax.fori_loop(0, NT // 2, step2, 0)
    drain_out(0, NT - 2)
    drain_out(1, NT - 1)


@jax.jit
def _sc_bilinear(u_r, v_r, fm_t):
    mesh = plsc.VectorSubcoreMesh(core_axis_name="c", subcore_axis_name="s")
    f = functools.partial(
        pl.kernel,
        mesh=mesh,
        out_type=jax.ShapeDtypeStruct((N_PAD, C), jnp.float32),
        scratch_types=[
            pltpu.VMEM((PW,), jnp.float32),
            pltpu.VMEM((PW,), jnp.float32),
            pltpu.VMEM((4 * B,), jnp.int32),
            pltpu.VMEM((4 * B,), jnp.int32),
            pltpu.VMEM((4 * B, C), jnp.float32),
            pltpu.VMEM((4 * B, C), jnp.float32),
            pltpu.SemaphoreType.DMA((2,)),
            pltpu.SemaphoreType.DMA((2,)),
        ],
    )(_body)
    return f(u_r, v_r, fm_t)


def kernel(feature_map, target_uv):
    fm_t = jnp.transpose(feature_map, (1, 2, 0)).reshape(H * W, C)
    n = target_uv.shape[0]
    uv = jnp.pad(target_uv, ((0, N_PAD - n), (0, 0)))
    uvr = uv.reshape(NT, NW, B, 2).transpose(1, 0, 2, 3).reshape(NW, PW, 2)
    return _sc_bilinear(uvr[:, :, 0], uvr[:, :, 1], fm_t)[:n].T


# speedup vs baseline: 1.0888x; 1.0888x over previous
import functools

import jax
import jax.numpy as jnp
from jax import lax
from jax.experimental import pallas as pl
from jax.experimental.pallas import tpu as pltpu
from jax.experimental.pallas import tpu_sc as plsc

C, H, W = 128, 512, 512
N = 200000
NW = 32
B = 96
NT = 66
NCHUNK = NW * NT
N_PAD = NCHUNK * B
PW = NT * B
G = B // 16


def _body(ur_hbm, vr_hbm, fm_hbm, out_hbm,
          u_all, v_all, i4a, i4b, r4a, r4b, gsem, osem):
    cid = lax.axis_index("c")
    sid = lax.axis_index("s")
    wid = sid * 2 + cid
    pltpu.sync_copy(ur_hbm.at[wid], u_all)
    pltpu.sync_copy(vr_hbm.at[wid], v_all)

    i4 = (i4a, i4b)
    r4 = (r4a, r4b)

    def fire_gather(b, k):
        koff = pl.multiple_of(k * B, B)
        for g in range(G):
            sl = pl.ds(koff + g * 16, 16)
            uu = u_all[sl]
            vv = v_all[sl]
            b00 = vv.astype(jnp.int32) * W + uu.astype(jnp.int32)
            i4[b][pl.ds(g * 16, 16)] = b00
            i4[b][pl.ds(B + g * 16, 16)] = b00 + 1
            i4[b][pl.ds(2 * B + g * 16, 16)] = b00 + W
            i4[b][pl.ds(3 * B + g * 16, 16)] = b00 + (W + 1)
        pltpu.async_copy(fm_hbm.at[i4[b]], r4[b], gsem.at[b])

    def combine_store(b, k):
        pltpu.make_async_copy(fm_hbm.at[i4[b]], r4[b], gsem.at[b]).wait()
        koff = pl.multiple_of(k * B, B)

        def gb(g, _):
            goff = pl.multiple_of(g * 16, 16)
            uu = u_all[pl.ds(koff + goff, 16)]
            vv = v_all[pl.ds(koff + goff, 16)]
            du = uu - uu.astype(jnp.int32).astype(jnp.float32)
            dv = vv - vv.astype(jnp.int32).astype(jnp.float32)
            omu = 1.0 - du
            omv = 1.0 - dv
            w00 = omv * omu
            w01 = omv * du
            w10 = dv * omu
            w11 = dv * du
            for j in range(16):
                s00 = jnp.full((16,), w00[j], jnp.float32)
                s01 = jnp.full((16,), w01[j], jnp.float32)
                s10 = jnp.full((16,), w10[j], jnp.float32)
                s11 = jnp.full((16,), w11[j], jnp.float32)
                p = goff + j
                for cg in range(C // 16):
                    cs = pl.ds(cg * 16, 16)
                    t0 = s00 * r4[b][p, cs] + s01 * r4[b][p + B, cs]
                    t1 = s10 * r4[b][p + 2 * B, cs] + s11 * r4[b][p + 3 * B, cs]
                    r4[b][p, cs] = t0 + t1
            return _

        lax.fori_loop(0, G, gb, 0)
        base = (wid + k * NW) * B
        pltpu.async_copy(r4[b].at[pl.ds(0, B)], out_hbm.at[pl.ds(base, B)],
                         osem.at[b])

    def drain_out(b, k):
        base = (wid + k * NW) * B
        pltpu.make_async_copy(r4[b].at[pl.ds(0, B)],
                              out_hbm.at[pl.ds(base, B)], osem.at[b]).wait()

    fire_gather(0, 0)

    def step2(m, carry):
        k0 = m * 2
        fire_gather(1, k0 + 1)

        @pl.when(m >= 1)
        def _():
            drain_out(0, k0 - 2)
        combine_store(0, k0)

        @pl.when(m + 1 <= NT // 2 - 1)
        def _():
            fire_gather(0, k0 + 2)

        @pl.when(m >= 1)
        def _():
            drain_out(1, k0 - 1)
        combine_store(1, k0 + 1)
        return carry

    lax.fori_loop(0, NT // 2, step2, 0)
    drain_out(0, NT - 2)
    drain_out(1, NT - 1)


@jax.jit
def _sc_bilinear(u_r, v_r, fm_t):
    mesh = plsc.VectorSubcoreMesh(core_axis_name="c", subcore_axis_name="s")
    f = functools.partial(
        pl.kernel,
        mesh=mesh,
        out_type=jax.ShapeDtypeStruct((N_PAD, C), jnp.float32),
        scratch_types=[
            pltpu.VMEM((PW,), jnp.float32),
            pltpu.VMEM((PW,), jnp.float32),
            pltpu.VMEM((4 * B,), jnp.int32),
            pltpu.VMEM((4 * B,), jnp.int32),
            pltpu.VMEM((4 * B, C), jnp.float32),
            pltpu.VMEM((4 * B, C), jnp.float32),
            pltpu.SemaphoreType.DMA((2,)),
            pltpu.SemaphoreType.DMA((2,)),
        ],
    )(_body)
    return f(u_r, v_r, fm_t)


def kernel(feature_map, target_uv):
    fm_t = jnp.transpose(feature_map, (1, 2, 0)).reshape(H * W, C)
    n = target_uv.shape[0]
    uv = jnp.pad(target_uv, ((0, N_PAD - n), (0, 0)))
    uvr = uv.reshape(NT, NW, B, 2).transpose(1, 0, 2, 3).reshape(NW, PW, 2)
    return _sc_bilinear(uvr[:, :, 0], uvr[:, :, 1], fm_t)[:n].T
